# baseline (speedup 1.0000x reference)
import functools

import jax
import jax.numpy as jnp
from jax import lax
from jax.experimental import pallas as pl
from jax.experimental.pallas import tpu as pltpu
from jax.experimental.pallas import tpu_sc as plsc

_N, _E, _NF, _FE, _D = 10000, 320000, 2000, 20000, 128
_NW = 32
_NCH = 80
_HALF = 5120
_ACCR = 5248
_ZRZ = 328
_ZRD = 320
_NACC = 10240
_NACCF = 2048
_ZRF = 128
_ACH = 3
_FCH = 5


def _sc_mesh():
  return plsc.VectorSubcoreMesh(
      core_axis_name="c", subcore_axis_name="s", num_cores=2, num_subcores=16)


def _atom_scatter(y, srcI, tgtI2, zeros):
  @functools.partial(
      pl.kernel,
      mesh=_sc_mesh(),
      out_type=jax.ShapeDtypeStruct((_NACC, _D), jnp.float32),
      scratch_types=[
          pltpu.VMEM((2 * _NCH, 128), jnp.int32),
          pltpu.VMEM((2 * _NCH, 128), jnp.int32),
          pltpu.VMEM((128, _D), jnp.float32),
          pltpu.VMEM((128, _D), jnp.float32),
          pltpu.VMEM((40, _D), jnp.float32),
          pltpu.VMEM_SHARED((_ACCR, _D), jnp.float32),
          pltpu.SemaphoreType.DMA,
          pltpu.SemaphoreType.DMA,
      ],
  )
  def k(y_hbm, srcI_hbm, tgtI2_hbm, zero_hbm, out_hbm,
        src_v, tgt_v, gbuf0, gbuf1, dbuf, acc_sh, gsem0, gsem1):
    c = lax.axis_index("c")
    s = lax.axis_index("s")
    pltpu.sync_copy(zero_hbm, dbuf)
    for q in range(8):
      pltpu.sync_copy(dbuf, acc_sh.at[pl.ds(s * _ZRZ + q * 40, 40)])
    pltpu.sync_copy(dbuf.at[pl.ds(0, 8)],
                    acc_sh.at[pl.ds(s * _ZRZ + 320, 8)])
    plsc.subcore_barrier()

    gb = (gbuf0, gbuf1)
    gs = (gsem0, gsem1)
    U = 8
    pltpu.sync_copy(srcI_hbm.at[s], src_v.at[pl.ds(0, _NCH)])
    pltpu.sync_copy(srcI_hbm.at[s + 16], src_v.at[pl.ds(_NCH, _NCH)])
    pltpu.sync_copy(tgtI2_hbm.at[c].at[s], tgt_v.at[pl.ds(0, _NCH)])
    pltpu.sync_copy(tgtI2_hbm.at[c].at[s + 16], tgt_v.at[pl.ds(_NCH, _NCH)])

    @pl.loop(0, 2 * _NCH // U)
    def step(i):
      base = i * U
      descs = [pltpu.async_copy(y_hbm.at[src_v.at[base]], gbuf0, gsem0)]
      for u in range(U):
        if u + 1 < U:
          descs.append(pltpu.async_copy(
              y_hbm.at[src_v.at[base + u + 1]], gb[(u + 1) % 2],
              gs[(u + 1) % 2]))
        descs[u].wait()
        pltpu.sync_copy(gb[u % 2], acc_sh.at[tgt_v.at[base + u]], add=True)

    plsc.subcore_barrier()
    for q in range(8):
      pltpu.sync_copy(acc_sh.at[pl.ds(s * _ZRD + q * 40, 40)], dbuf)
      pltpu.sync_copy(
          dbuf, out_hbm.at[pl.ds(c * _HALF + s * _ZRD + q * 40, 40)])

  return k(y, srcI, tgtI2, zeros)


def _frag_scatter(table, srcI, tgtI, zeros, *, n_ch):
  @functools.partial(
      pl.kernel,
      mesh=_sc_mesh(),
      out_type=jax.ShapeDtypeStruct((2, _NACCF, _D), jnp.float32),
      scratch_types=[
          pltpu.VMEM((n_ch, 128), jnp.int32),
          pltpu.VMEM((n_ch, 128), jnp.int32),
          pltpu.VMEM((128, _D), jnp.float32),
          pltpu.VMEM((_ZRF, _D), jnp.float32),
          pltpu.VMEM_SHARED((_NACCF, _D), jnp.float32),
      ],
  )
  def k(table_hbm, srcI_hbm, tgtI_hbm, zero_hbm, out_hbm,
        src_v, tgt_v, gbuf, dbuf, acc_sh):
    c = lax.axis_index("c")
    s = lax.axis_index("s")
    wid = c * 16 + s
    pltpu.sync_copy(srcI_hbm.at[wid], src_v)
    pltpu.sync_copy(tgtI_hbm.at[wid], tgt_v)
    pltpu.sync_copy(zero_hbm, dbuf)
    pltpu.sync_copy(dbuf, acc_sh.at[pl.ds(s * _ZRF, _ZRF)])
    plsc.subcore_barrier()

    @pl.loop(0, n_ch)
    def step(j):
      pltpu.sync_copy(table_hbm.at[src_v.at[j]], gbuf)
      pltpu.sync_copy(gbuf, acc_sh.at[tgt_v.at[j]], add=True)

    plsc.subcore_barrier()
    pltpu.sync_copy(acc_sh.at[pl.ds(s * _ZRF, _ZRF)], dbuf)
    pltpu.sync_copy(dbuf, out_hbm.at[c].at[pl.ds(s * _ZRF, _ZRF)])

  return k(table, srcI, tgtI, zeros)


def _degree(degI2, ones, zeros):
  @functools.partial(
      pl.kernel,
      mesh=_sc_mesh(),
      out_type=jax.ShapeDtypeStruct((_NACC, _D), jnp.float32),
      scratch_types=[
          pltpu.VMEM((_NCH, 128), jnp.int32),
          pltpu.VMEM((128, _D), jnp.float32),
          pltpu.VMEM((40, _D), jnp.float32),
          pltpu.VMEM_SHARED((_ACCR, _D), jnp.float32),
      ],
  )
  def k(degI2_hbm, ones_hbm, zero_hbm, out_hbm, idx_v, vbuf, dbuf, acc_sh):
    c = lax.axis_index("c")
    s = lax.axis_index("s")
    pltpu.sync_copy(ones_hbm, vbuf)
    pltpu.sync_copy(zero_hbm, dbuf)
    for q in range(8):
      pltpu.sync_copy(dbuf, acc_sh.at[pl.ds(s * _ZRZ + q * 40, 40)])
    pltpu.sync_copy(dbuf.at[pl.ds(0, 8)],
                    acc_sh.at[pl.ds(s * _ZRZ + 320, 8)])
    plsc.subcore_barrier()

    for half in range(2):
      blk = half * 16 + s
      pltpu.sync_copy(degI2_hbm.at[c].at[blk], idx_v)

      @pl.loop(0, _NCH)
      def step(j):
        pltpu.sync_copy(vbuf, acc_sh.at[idx_v.at[j]], add=True)

    plsc.subcore_barrier()
    for q in range(8):
      pltpu.sync_copy(acc_sh.at[pl.ds(s * _ZRD + q * 40, 40)], dbuf)
      pltpu.sync_copy(
          dbuf, out_hbm.at[pl.ds(c * _HALF + s * _ZRD + q * 40, 40)])

  return k(degI2, ones, zeros)


_BLK = 1000


def _prep_dinv(degp):
  def body(dp_ref, out_ref):
    out_ref[...] = lax.rsqrt(dp_ref[:, 0:1] + 1.0)

  blk = _NACC // 8
  out = pl.pallas_call(
      body,
      grid=(8,),
      in_specs=[pl.BlockSpec((blk, _D), lambda i: (i, 0))],
      out_specs=pl.BlockSpec((blk, 1), lambda i: (i, 0)),
      out_shape=jax.ShapeDtypeStruct((_NACC, 1), jnp.float32),
  )(degp)
  return out[:_N]


def _mm_first(x, dinv, W, b):
  def body(x_ref, dinv_ref, W_ref, b_ref, out_ref):
    acc = jnp.dot(x_ref[...], W_ref[...], preferred_element_type=jnp.float32)
    out_ref[...] = dinv_ref[...] * (acc + b_ref[...])

  return pl.pallas_call(
      body,
      grid=(_N // _BLK,),
      in_specs=[
          pl.BlockSpec((_BLK, _D), lambda i: (i, 0)),
          pl.BlockSpec((_BLK, 1), lambda i: (i, 0)),
          pl.BlockSpec((_D, _D), lambda i: (0, 0)),
          pl.BlockSpec((1, _D), lambda i: (0, 0)),
      ],
      out_specs=pl.BlockSpec((_BLK, _D), lambda i: (i, 0)),
      out_shape=jax.ShapeDtypeStruct((_N, _D), jnp.float32),
  )(x, dinv, W, b)


def _mm_mid(z, y, dinv, W, b):
  def body(z_ref, y_ref, dinv_ref, W_ref, b_ref, out_ref):
    pre = dinv_ref[...] * (z_ref[...] + y_ref[...])
    x = jnp.maximum(pre, 0.0)
    acc = jnp.dot(x, W_ref[...], preferred_element_type=jnp.float32)
    out_ref[...] = dinv_ref[...] * (acc + b_ref[...])

  return pl.pallas_call(
      body,
      grid=(_N // _BLK,),
      in_specs=[
          pl.BlockSpec((_BLK, _D), lambda i: (i, 0)),
          pl.BlockSpec((_BLK, _D), lambda i: (i, 0)),
          pl.BlockSpec((_BLK, 1), lambda i: (i, 0)),
          pl.BlockSpec((_D, _D), lambda i: (0, 0)),
          pl.BlockSpec((1, _D), lambda i: (0, 0)),
      ],
      out_specs=pl.BlockSpec((_BLK, _D), lambda i: (i, 0)),
      out_shape=jax.ShapeDtypeStruct((_N, _D), jnp.float32),
  )(z, y, dinv, W, b)


def _final_atoms(z, y, dinv):
  def body(z_ref, y_ref, dinv_ref, pre_ref, out_ref):
    pre = dinv_ref[...] * (z_ref[...] + y_ref[...])
    pre_ref[...] = pre
    out_ref[...] = jnp.maximum(pre, 0.0)

  return pl.pallas_call(
      body,
      grid=(_N // _BLK,),
      in_specs=[
          pl.BlockSpec((_BLK, _D), lambda i: (i, 0)),
          pl.BlockSpec((_BLK, _D), lambda i: (i, 0)),
          pl.BlockSpec((_BLK, 1), lambda i: (i, 0)),
      ],
      out_specs=[
          pl.BlockSpec((_BLK, _D), lambda i: (i, 0)),
          pl.BlockSpec((_BLK, _D), lambda i: (i, 0)),
      ],
      out_shape=[
          jax.ShapeDtypeStruct((_N, _D), jnp.float32),
          jax.ShapeDtypeStruct((_N, _D), jnp.float32),
      ],
  )(z, y, dinv)


def _frag_add(fp):
  def body(fp_ref, out_ref):
    out_ref[...] = fp_ref[0] + fp_ref[1]

  blk = _NACCF // 2
  return pl.pallas_call(
      body,
      grid=(2,),
      in_specs=[pl.BlockSpec((2, blk, _D), lambda i: (0, i, 0))],
      out_specs=pl.BlockSpec((blk, _D), lambda i: (i, 0)),
      out_shape=jax.ShapeDtypeStruct((_NACCF, _D), jnp.float32),
  )(fp)


def _frag_mlp(fp, W1, b1, W2, b2):
  def body(fp_ref, W1_ref, b1_ref, W2_ref, b2_ref, out_ref):
    ffs = fp_ref[0] + fp_ref[1]
    h = jnp.maximum(
        jnp.dot(ffs, W1_ref[...], preferred_element_type=jnp.float32)
        + b1_ref[...], 0.0)
    out_ref[...] = jnp.maximum(
        jnp.dot(h, W2_ref[...], preferred_element_type=jnp.float32)
        + b2_ref[...], 0.0)

  blk = _NACCF // 2
  out = pl.pallas_call(
      body,
      grid=(2,),
      in_specs=[
          pl.BlockSpec((2, blk, _D), lambda i: (0, i, 0)),
          pl.BlockSpec((_D, 2 * _D), lambda i: (0, 0)),
          pl.BlockSpec((1, 2 * _D), lambda i: (0, 0)),
          pl.BlockSpec((2 * _D, _D), lambda i: (0, 0)),
          pl.BlockSpec((1, _D), lambda i: (0, 0)),
      ],
      out_specs=pl.BlockSpec((blk, _D), lambda i: (i, 0)),
      out_shape=jax.ShapeDtypeStruct((_NACCF, _D), jnp.float32),
  )(fp, W1, b1, W2, b2)
  return out[:_NF]


def kernel(x_atoms, edge_index, edge_attr, frag_index, x_frags, batch,
           frag_batch, atom_to_frag_ids, node_features_bonds,
           edge_index_bonds_graph, edge_attr_bonds,
           l1_atom_W, l1_atom_b, l1_edge_W, l1_edge_b, l1_fm_W1, l1_fm_b1, l1_fm_W2, l1_fm_b2,
           l2_atom_W, l2_atom_b, l2_edge_W, l2_edge_b, l2_fm_W1, l2_fm_b1, l2_fm_W2, l2_fm_b2,
           l3_atom_W, l3_atom_b, l3_edge_W, l3_edge_b, l3_fm_W1, l3_fm_b1, l3_fm_W2, l3_fm_b2,
           l4_atom_W, l4_atom_b, l4_edge_W, l4_edge_b, l4_fm_W1, l4_fm_b1, l4_fm_W2, l4_fm_b2):
  src = edge_index[0]
  tgt = edge_index[1]
  ep = _NW * _NCH * 128
  epad = ep - _E
  srcG = jnp.concatenate(
      [src, jnp.zeros((epad,), jnp.int32)]).reshape(_NW, _NCH, 128)
  dsp = _HALF + (jnp.arange(ep, dtype=jnp.int32) % 128)
  tgt_p = jnp.concatenate([tgt, jnp.full((epad,), _N, jnp.int32)])
  tA = jnp.where(tgt_p < _HALF, tgt_p, dsp)
  tB = jnp.where((tgt_p >= _HALF) & (tgt_p < _N), tgt_p - _HALF, dsp)
  tgtI2 = jnp.stack([tA, tB]).reshape(2, _NW, _NCH, 128)
  src_p = jnp.concatenate([src, jnp.full((epad,), _N, jnp.int32)])
  dA = jnp.where(src_p < _HALF, src_p, dsp)
  dB = jnp.where((src_p >= _HALF) & (src_p < _N), src_p - _HALF, dsp)
  degI2 = jnp.stack([dA, dB]).reshape(2, _NW, _NCH, 128)

  apad = _NW * _ACH * 128 - _N
  aidxG = jnp.concatenate(
      [jnp.arange(_N, dtype=jnp.int32), jnp.zeros((apad,), jnp.int32)]
  ).reshape(_NW, _ACH, 128)
  a2fG = jnp.concatenate(
      [atom_to_frag_ids.astype(jnp.int32),
       _NF + (jnp.arange(apad, dtype=jnp.int32) % 48)]
  ).reshape(_NW, _ACH, 128)

  fpad = _NW * _FCH * 128 - _FE
  fsrcG = jnp.concatenate(
      [frag_index[0], jnp.zeros((fpad,), jnp.int32)]).reshape(_NW, _FCH, 128)
  ftgtG = jnp.concatenate(
      [frag_index[1], _NF + (jnp.arange(fpad, dtype=jnp.int32) % 48)]
  ).reshape(_NW, _FCH, 128)

  zerosA = jnp.zeros((40, _D), jnp.float32)
  zerosF = jnp.zeros((_ZRF, _D), jnp.float32)
  onesD = jnp.zeros((128, _D), jnp.float32).at[:, 0].set(1.0)

  degp = _degree(degI2, onesD, zerosA)
  dinv = _prep_dinv(degp)

  Ws = (l1_atom_W, l2_atom_W, l3_atom_W, l4_atom_W)
  bs = (l1_atom_b.reshape(1, _D), l2_atom_b.reshape(1, _D),
        l3_atom_b.reshape(1, _D), l4_atom_b.reshape(1, _D))

  y = _mm_first(x_atoms, dinv, Ws[0], bs[0])
  for i in (1, 2, 3):
    z = _atom_scatter(y, srcG, tgtI2, zerosA)
    y = _mm_mid(z, y, dinv, Ws[i], bs[i])
  z = _atom_scatter(y, srcG, tgtI2, zerosA)
  pre4, xa = _final_atoms(z, y, dinv)

  fpp = _frag_scatter(pre4, aidxG, a2fG, zerosF, n_ch=_ACH)
  xfr = _frag_add(fpp)
  ffp = _frag_scatter(xfr, fsrcG, ftgtG, zerosF, n_ch=_FCH)
  xf = _frag_mlp(ffp, l4_fm_W1, l4_fm_b1.reshape(1, 2 * _D),
                 l4_fm_W2, l4_fm_b2.reshape(1, _D))
  return xa, xf


# speedup vs baseline: 1.3124x; 1.3124x over previous
import functools

import jax
import jax.numpy as jnp
from jax import lax
from jax.experimental import pallas as pl
from jax.experimental.pallas import tpu as pltpu
from jax.experimental.pallas import tpu_sc as plsc

_N, _E, _NF, _FE, _D = 10000, 320000, 2000, 20000, 128
_NW = 32
_NCH = 79
_HALF = 5120
_ACCR = 5248
_ZRZ = 328
_ZRD = 320
_NACC = 10240
_NACCF = 2048
_ZRF = 128
_ACH = 3
_FCH = 5


def _sc_mesh():
  return plsc.VectorSubcoreMesh(
      core_axis_name="c", subcore_axis_name="s", num_cores=2, num_subcores=16)


def _atom_scatter(y, srcI, tgtI2, zeros):
  @functools.partial(
      pl.kernel,
      mesh=_sc_mesh(),
      out_type=jax.ShapeDtypeStruct((_NACC, _D), jnp.float32),
      scratch_types=[
          pltpu.VMEM((_NCH, 128), jnp.int32),
          pltpu.VMEM((_NCH, 128), jnp.int32),
          pltpu.VMEM((128, _D), jnp.float32),
          pltpu.VMEM((_ZRZ, _D), jnp.float32),
          pltpu.VMEM_SHARED((_ACCR, _D), jnp.float32),
      ],
  )
  def k(y_hbm, srcI_hbm, tgtI2_hbm, zero_hbm, out_hbm,
        src_v, tgt_v, gbuf, dbuf, acc_sh):
    c = lax.axis_index("c")
    s = lax.axis_index("s")
    pltpu.sync_copy(zero_hbm, dbuf)
    pltpu.sync_copy(dbuf, acc_sh.at[pl.ds(s * _ZRZ, _ZRZ)])
    plsc.subcore_barrier()

    for half in range(2):
      blk = half * 16 + s
      pltpu.sync_copy(srcI_hbm.at[blk], src_v)
      pltpu.sync_copy(tgtI2_hbm.at[c].at[blk], tgt_v)

      @pl.loop(0, _NCH)
      def step(j):
        pltpu.sync_copy(y_hbm.at[src_v.at[j]], gbuf)
        pltpu.sync_copy(gbuf, acc_sh.at[tgt_v.at[j]], add=True)

    plsc.subcore_barrier()
    pltpu.sync_copy(acc_sh.at[pl.ds(s * _ZRD, _ZRD)], dbuf.at[pl.ds(0, _ZRD)])
    pltpu.sync_copy(dbuf.at[pl.ds(0, _ZRD)],
                    out_hbm.at[pl.ds(c * _HALF + s * _ZRD, _ZRD)])

  return k(y, srcI, tgtI2, zeros)


def _frag_scatter(table, srcI, tgtI, zeros, *, n_ch):
  @functools.partial(
      pl.kernel,
      mesh=_sc_mesh(),
      out_type=jax.ShapeDtypeStruct((2, _NACCF, _D), jnp.float32),
      scratch_types=[
          pltpu.VMEM((n_ch, 128), jnp.int32),
          pltpu.VMEM((n_ch, 128), jnp.int32),
          pltpu.VMEM((128, _D), jnp.float32),
          pltpu.VMEM((_ZRF, _D), jnp.float32),
          pltpu.VMEM_SHARED((_NACCF, _D), jnp.float32),
      ],
  )
  def k(table_hbm, srcI_hbm, tgtI_hbm, zero_hbm, out_hbm,
        src_v, tgt_v, gbuf, dbuf, acc_sh):
    c = lax.axis_index("c")
    s = lax.axis_index("s")
    wid = c * 16 + s
    pltpu.sync_copy(srcI_hbm.at[wid], src_v)
    pltpu.sync_copy(tgtI_hbm.at[wid], tgt_v)
    pltpu.sync_copy(zero_hbm, dbuf)
    pltpu.sync_copy(dbuf, acc_sh.at[pl.ds(s * _ZRF, _ZRF)])
    plsc.subcore_barrier()

    @pl.loop(0, n_ch)
    def step(j):
      pltpu.sync_copy(table_hbm.at[src_v.at[j]], gbuf)
      pltpu.sync_copy(gbuf, acc_sh.at[tgt_v.at[j]], add=True)

    plsc.subcore_barrier()
    pltpu.sync_copy(acc_sh.at[pl.ds(s * _ZRF, _ZRF)], dbuf)
    pltpu.sync_copy(dbuf, out_hbm.at[c].at[pl.ds(s * _ZRF, _ZRF)])

  return k(table, srcI, tgtI, zeros)


def _degree(degI2, ones, zeros):
  @functools.partial(
      pl.kernel,
      mesh=_sc_mesh(),
      out_type=jax.ShapeDtypeStruct((_NACC, _D), jnp.float32),
      scratch_types=[
          pltpu.VMEM((_NCH, 128), jnp.int32),
          pltpu.VMEM((128, _D), jnp.float32),
          pltpu.VMEM((_ZRZ, _D), jnp.float32),
          pltpu.VMEM_SHARED((_ACCR, _D), jnp.float32),
      ],
  )
  def k(degI2_hbm, ones_hbm, zero_hbm, out_hbm, idx_v, vbuf, dbuf, acc_sh):
    c = lax.axis_index("c")
    s = lax.axis_index("s")
    pltpu.sync_copy(ones_hbm, vbuf)
    pltpu.sync_copy(zero_hbm, dbuf)
    pltpu.sync_copy(dbuf, acc_sh.at[pl.ds(s * _ZRZ, _ZRZ)])
    plsc.subcore_barrier()

    for half in range(2):
      blk = half * 16 + s
      pltpu.sync_copy(degI2_hbm.at[c].at[blk], idx_v)

      @pl.loop(0, _NCH)
      def step(j):
        pltpu.sync_copy(vbuf, acc_sh.at[idx_v.at[j]], add=True)

    plsc.subcore_barrier()
    pltpu.sync_copy(acc_sh.at[pl.ds(s * _ZRD, _ZRD)], dbuf.at[pl.ds(0, _ZRD)])
    pltpu.sync_copy(dbuf.at[pl.ds(0, _ZRD)],
                    out_hbm.at[pl.ds(c * _HALF + s * _ZRD, _ZRD)])

  return k(degI2, ones, zeros)


_BLK = 1000


def _prep_dinv(degp):
  def body(dp_ref, out_ref):
    out_ref[...] = lax.rsqrt(dp_ref[:, 0:1] + 1.0)

  blk = _NACC // 8
  out = pl.pallas_call(
      body,
      grid=(8,),
      in_specs=[pl.BlockSpec((blk, _D), lambda i: (i, 0))],
      out_specs=pl.BlockSpec((blk, 1), lambda i: (i, 0)),
      out_shape=jax.ShapeDtypeStruct((_NACC, 1), jnp.float32),
  )(degp)
  return out[:_N]


def _mm_first(x, dinv, W, b):
  def body(x_ref, dinv_ref, W_ref, b_ref, out_ref):
    acc = jnp.dot(x_ref[...], W_ref[...], preferred_element_type=jnp.float32)
    out_ref[...] = dinv_ref[...] * (acc + b_ref[...])

  return pl.pallas_call(
      body,
      grid=(_N // _BLK,),
      in_specs=[
          pl.BlockSpec((_BLK, _D), lambda i: (i, 0)),
          pl.BlockSpec((_BLK, 1), lambda i: (i, 0)),
          pl.BlockSpec((_D, _D), lambda i: (0, 0)),
          pl.BlockSpec((1, _D), lambda i: (0, 0)),
      ],
      out_specs=pl.BlockSpec((_BLK, _D), lambda i: (i, 0)),
      out_shape=jax.ShapeDtypeStruct((_N, _D), jnp.float32),
  )(x, dinv, W, b)


def _mm_mid(z, y, dinv, W, b):
  def body(z_ref, y_ref, dinv_ref, W_ref, b_ref, out_ref):
    pre = dinv_ref[...] * (z_ref[...] + y_ref[...])
    x = jnp.maximum(pre, 0.0)
    acc = jnp.dot(x, W_ref[...], preferred_element_type=jnp.float32)
    out_ref[...] = dinv_ref[...] * (acc + b_ref[...])

  return pl.pallas_call(
      body,
      grid=(_N // _BLK,),
      in_specs=[
          pl.BlockSpec((_BLK, _D), lambda i: (i, 0)),
          pl.BlockSpec((_BLK, _D), lambda i: (i, 0)),
          pl.BlockSpec((_BLK, 1), lambda i: (i, 0)),
          pl.BlockSpec((_D, _D), lambda i: (0, 0)),
          pl.BlockSpec((1, _D), lambda i: (0, 0)),
      ],
      out_specs=pl.BlockSpec((_BLK, _D), lambda i: (i, 0)),
      out_shape=jax.ShapeDtypeStruct((_N, _D), jnp.float32),
  )(z, y, dinv, W, b)


def _final_atoms(z, y, dinv):
  def body(z_ref, y_ref, dinv_ref, pre_ref, out_ref):
    pre = dinv_ref[...] * (z_ref[...] + y_ref[...])
    pre_ref[...] = pre
    out_ref[...] = jnp.maximum(pre, 0.0)

  return pl.pallas_call(
      body,
      grid=(_N // _BLK,),
      in_specs=[
          pl.BlockSpec((_BLK, _D), lambda i: (i, 0)),
          pl.BlockSpec((_BLK, _D), lambda i: (i, 0)),
          pl.BlockSpec((_BLK, 1), lambda i: (i, 0)),
      ],
      out_specs=[
          pl.BlockSpec((_BLK, _D), lambda i: (i, 0)),
          pl.BlockSpec((_BLK, _D), lambda i: (i, 0)),
      ],
      out_shape=[
          jax.ShapeDtypeStruct((_N, _D), jnp.float32),
          jax.ShapeDtypeStruct((_N, _D), jnp.float32),
      ],
  )(z, y, dinv)


def _frag_add(fp):
  def body(fp_ref, out_ref):
    out_ref[...] = fp_ref[0] + fp_ref[1]

  blk = _NACCF // 2
  return pl.pallas_call(
      body,
      grid=(2,),
      in_specs=[pl.BlockSpec((2, blk, _D), lambda i: (0, i, 0))],
      out_specs=pl.BlockSpec((blk, _D), lambda i: (i, 0)),
      out_shape=jax.ShapeDtypeStruct((_NACCF, _D), jnp.float32),
  )(fp)


def _frag_mlp(fp, W1, b1, W2, b2):
  def body(fp_ref, W1_ref, b1_ref, W2_ref, b2_ref, out_ref):
    ffs = fp_ref[0] + fp_ref[1]
    h = jnp.maximum(
        jnp.dot(ffs, W1_ref[...], preferred_element_type=jnp.float32)
        + b1_ref[...], 0.0)
    out_ref[...] = jnp.maximum(
        jnp.dot(h, W2_ref[...], preferred_element_type=jnp.float32)
        + b2_ref[...], 0.0)

  blk = _NACCF // 2
  out = pl.pallas_call(
      body,
      grid=(2,),
      in_specs=[
          pl.BlockSpec((2, blk, _D), lambda i: (0, i, 0)),
          pl.BlockSpec((_D, 2 * _D), lambda i: (0, 0)),
          pl.BlockSpec((1, 2 * _D), lambda i: (0, 0)),
          pl.BlockSpec((2 * _D, _D), lambda i: (0, 0)),
          pl.BlockSpec((1, _D), lambda i: (0, 0)),
      ],
      out_specs=pl.BlockSpec((blk, _D), lambda i: (i, 0)),
      out_shape=jax.ShapeDtypeStruct((_NACCF, _D), jnp.float32),
  )(fp, W1, b1, W2, b2)
  return out[:_NF]


def kernel(x_atoms, edge_index, edge_attr, frag_index, x_frags, batch,
           frag_batch, atom_to_frag_ids, node_features_bonds,
           edge_index_bonds_graph, edge_attr_bonds,
           l1_atom_W, l1_atom_b, l1_edge_W, l1_edge_b, l1_fm_W1, l1_fm_b1, l1_fm_W2, l1_fm_b2,
           l2_atom_W, l2_atom_b, l2_edge_W, l2_edge_b, l2_fm_W1, l2_fm_b1, l2_fm_W2, l2_fm_b2,
           l3_atom_W, l3_atom_b, l3_edge_W, l3_edge_b, l3_fm_W1, l3_fm_b1, l3_fm_W2, l3_fm_b2,
           l4_atom_W, l4_atom_b, l4_edge_W, l4_edge_b, l4_fm_W1, l4_fm_b1, l4_fm_W2, l4_fm_b2):
  src = edge_index[0]
  tgt = edge_index[1]
  ep = _NW * _NCH * 128
  epad = ep - _E
  srcG = jnp.concatenate(
      [src, jnp.zeros((epad,), jnp.int32)]).reshape(_NW, _NCH, 128)
  dsp = _HALF + (jnp.arange(ep, dtype=jnp.int32) % 128)
  tgt_p = jnp.concatenate([tgt, jnp.full((epad,), _N, jnp.int32)])
  tA = jnp.where(tgt_p < _HALF, tgt_p, dsp)
  tB = jnp.where((tgt_p >= _HALF) & (tgt_p < _N), tgt_p - _HALF, dsp)
  tgtI2 = jnp.stack([tA, tB]).reshape(2, _NW, _NCH, 128)
  src_p = jnp.concatenate([src, jnp.full((epad,), _N, jnp.int32)])
  dA = jnp.where(src_p < _HALF, src_p, dsp)
  dB = jnp.where((src_p >= _HALF) & (src_p < _N), src_p - _HALF, dsp)
  degI2 = jnp.stack([dA, dB]).reshape(2, _NW, _NCH, 128)

  apad = _NW * _ACH * 128 - _N
  aidxG = jnp.concatenate(
      [jnp.arange(_N, dtype=jnp.int32), jnp.zeros((apad,), jnp.int32)]
  ).reshape(_NW, _ACH, 128)
  a2fG = jnp.concatenate(
      [atom_to_frag_ids.astype(jnp.int32),
       _NF + (jnp.arange(apad, dtype=jnp.int32) % 48)]
  ).reshape(_NW, _ACH, 128)

  fpad = _NW * _FCH * 128 - _FE
  fsrcG = jnp.concatenate(
      [frag_index[0], jnp.zeros((fpad,), jnp.int32)]).reshape(_NW, _FCH, 128)
  ftgtG = jnp.concatenate(
      [frag_index[1], _NF + (jnp.arange(fpad, dtype=jnp.int32) % 48)]
  ).reshape(_NW, _FCH, 128)

  zerosA = jnp.zeros((_ZRZ, _D), jnp.float32)
  zerosF = jnp.zeros((_ZRF, _D), jnp.float32)
  onesD = jnp.zeros((128, _D), jnp.float32).at[:, 0].set(1.0)

  degp = _degree(degI2, onesD, zerosA)
  dinv = _prep_dinv(degp)

  Ws = (l1_atom_W, l2_atom_W, l3_atom_W, l4_atom_W)
  bs = (l1_atom_b.reshape(1, _D), l2_atom_b.reshape(1, _D),
        l3_atom_b.reshape(1, _D), l4_atom_b.reshape(1, _D))

  y = _mm_first(x_atoms, dinv, Ws[0], bs[0])
  for i in (1, 2, 3):
    z = _atom_scatter(y, srcG, tgtI2, zerosA)
    y = _mm_mid(z, y, dinv, Ws[i], bs[i])
  z = _atom_scatter(y, srcG, tgtI2, zerosA)
  pre4, xa = _final_atoms(z, y, dinv)

  fpp = _frag_scatter(pre4, aidxG, a2fG, zerosF, n_ch=_ACH)
  xfr = _frag_add(fpp)
  ffp = _frag_scatter(xfr, fsrcG, ftgtG, zerosF, n_ch=_FCH)
  xf = _frag_mlp(ffp, l4_fm_W1, l4_fm_b1.reshape(1, 2 * _D),
                 l4_fm_W2, l4_fm_b2.reshape(1, _D))
  return xa, xf
